# baseline (speedup 1.0000x reference)
import dataclasses
import functools

import jax
import jax.numpy as jnp
from jax import lax
from jax.experimental import pallas as pl
from jax.experimental.pallas import tpu as pltpu
from jax.experimental.pallas import tpu_sc as plsc

N = 10000
NC = 32
EC = 128
HC = 64
NK = 5
R2 = 0.2 * 0.2
KS = 32
STEPS = 5

FK = NK * HC
NP = 10240
CP = 10240
NTILES = 32
RPT = NP // NTILES
CAP = 512
CM = CAP // 16
NCHUNK = CP // 16
BIG = 1e9


def _d2_body(a_ref, b_ref, out_ref):
    i = pl.program_id(0)
    j = pl.program_id(1)
    rb = out_ref.shape[0]
    cb = out_ref.shape[1]
    acc = None
    for co in range(3):
        a = a_ref[:, co:co + 1]
        b = b_ref[co:co + 1, :]
        d = a - b
        acc = d * d if acc is None else acc + d * d
    rid = lax.broadcasted_iota(jnp.int32, (rb, cb), 0) + i * rb
    cid = lax.broadcasted_iota(jnp.int32, (rb, cb), 1) + j * cb
    out_ref[...] = jnp.where(rid == cid, BIG, acc)


def _d2_matrix(pos8, pos8t):
    rb, cb = 640, 2560
    return pl.pallas_call(
        _d2_body,
        grid=(NP // rb, CP // cb),
        in_specs=[
            pl.BlockSpec((rb, 8), lambda i, j: (i, 0)),
            pl.BlockSpec((8, cb), lambda i, j: (0, j)),
        ],
        out_specs=pl.BlockSpec((rb, cb), lambda i, j: (i, j)),
        out_shape=jax.ShapeDtypeStruct((NP, CP), jnp.float32),
    )(pos8, pos8t)


def _knn_select(d2):
    mesh = plsc.VectorSubcoreMesh(core_axis_name="c", subcore_axis_name="s")

    cp = pltpu.CompilerParams()
    if "needs_layout_passes" in pltpu.CompilerParams.__dataclass_fields__:
        cp = dataclasses.replace(cp, needs_layout_passes=False)

    @functools.partial(
        pl.kernel,
        compiler_params=cp,
        out_type=[
            jax.ShapeDtypeStruct((NP * KS,), jnp.int32),
            jax.ShapeDtypeStruct((NP * KS,), jnp.float32),
        ],
        mesh=mesh,
        scratch_types=[
            pltpu.VMEM((2, CP), jnp.float32),
            pltpu.VMEM((CAP,), jnp.float32),
            pltpu.VMEM((CAP,), jnp.int32),
            pltpu.VMEM((RPT * KS,), jnp.int32),
            pltpu.VMEM((RPT * KS,), jnp.float32),
            pltpu.SemaphoreType.DMA,
            pltpu.SemaphoreType.DMA,
        ],
    )
    def sel_kernel(d2_hbm, idx_hbm, val_hbm, rowbuf, cvals, cidx,
                   oidx, oval, sem_in, sem_out):
        wid = lax.axis_index("s") * 2 + lax.axis_index("c")
        base = wid * RPT
        iota16 = lax.iota(jnp.int32, 16)
        inf16 = jnp.full((16,), BIG, jnp.float32)
        zero16 = jnp.zeros((16,), jnp.int32)

        def bc(x):
            return jnp.broadcast_to(x, (16,))

        def process(buf, lrow):
            @pl.loop(0, CM)
            def _(jc):
                cvals[pl.ds(jc * 16, 16)] = inf16
                cidx[pl.ds(jc * 16, 16)] = iota16 + jc * 16

            UNR = 8

            def chunk_body(g, cntv):
                vs, ms, pss, pcs = [], [], [], []
                for uu in range(UNR):
                    c = g * UNR + uu
                    v = rowbuf[buf, pl.ds(c * 16, 16)]
                    m = v < R2
                    vs.append(v)
                    ms.append(m)
                    pss.append(jnp.cumsum(m.astype(jnp.int32)))
                    pcs.append(plsc.all_reduce_population_count(m))
                for uu in range(UNR):
                    c = g * UNR + uu
                    pos = cntv + pss[uu] - 1
                    mm = ms[uu] & (pos < CAP)
                    colv = iota16 + c * 16
                    plsc.store_scatter(cvals, [pos], vs[uu], mask=mm)
                    plsc.store_scatter(cidx, [pos], colv, mask=mm)
                    cntv = cntv + pcs[uu]
                return cntv

            lax.fori_loop(0, NCHUNK // UNR, chunk_body,
                          jnp.zeros((16,), jnp.int32))

            def cmin16(cbase):
                acc = inf16
                for l in range(16):
                    g = plsc.load_gather(cvals, [(iota16 + cbase) * 16 + l])
                    acc = jnp.minimum(acc, g)
                return acc

            cm0 = cmin16(0)
            cm1 = cmin16(16)

            def ext_body(t, carry):
                cm0, cm1, sel, selv = carry
                minv = jnp.min(jnp.minimum(cm0, cm1))
                p0 = jnp.where(cm0 == minv, iota16, 64)
                p1 = jnp.where(cm1 == minv, iota16 + 16, 64)
                cstar = jnp.min(jnp.minimum(p0, p1))
                off = cstar * 16
                v = cvals[pl.ds(off, 16)]
                lane = jnp.min(jnp.where(v == minv, iota16, 15))
                col = plsc.load_gather(cidx, [bc(off + lane)])
                validv = jnp.where(bc(minv) < R2, 1.0, 0.0)
                tl = t % 16
                sel = jnp.where(iota16 == tl, col, sel)
                selv = jnp.where(iota16 == tl, validv, selv)
                vupd = jnp.where(iota16 == lane, BIG, v)
                cvals[pl.ds(off, 16)] = vupd
                nm = bc(jnp.min(vupd))
                cm0 = jnp.where(iota16 == cstar, nm, cm0)
                cm1 = jnp.where(iota16 + 16 == cstar, nm, cm1)
                return cm0, cm1, sel, selv

            obase = lrow * KS
            cm0, cm1, sel, selv = lax.fori_loop(
                0, 16, ext_body, (cm0, cm1, zero16, jnp.zeros((16,), jnp.float32)))
            oidx[pl.ds(obase, 16)] = sel
            oval[pl.ds(obase, 16)] = selv
            cm0, cm1, sel, selv = lax.fori_loop(
                16, 32, ext_body, (cm0, cm1, zero16, jnp.zeros((16,), jnp.float32)))
            oidx[pl.ds(obase + 16, 16)] = sel
            oval[pl.ds(obase + 16, 16)] = selv

        pltpu.make_async_copy(d2_hbm.at[base], rowbuf.at[0], sem_in).start()
        pltpu.make_async_copy(d2_hbm.at[base + 1], rowbuf.at[1], sem_in).start()

        @pl.loop(0, RPT, step=2)
        def _(g):
            pltpu.make_async_copy(d2_hbm.at[base + g],
                                  rowbuf.at[0], sem_in).wait()
            process(0, g)

            @pl.when(g + 2 < RPT)
            def _():
                pltpu.make_async_copy(d2_hbm.at[base + g + 2],
                                      rowbuf.at[0], sem_in).start()

            pltpu.make_async_copy(d2_hbm.at[base + g + 1],
                                  rowbuf.at[1], sem_in).wait()
            process(1, g + 1)

            @pl.when(g + 3 < RPT)
            def _():
                pltpu.make_async_copy(d2_hbm.at[base + g + 3],
                                      rowbuf.at[1], sem_in).start()

        pltpu.make_async_copy(oidx, idx_hbm.at[pl.ds(base * KS, RPT * KS)],
                              sem_out).start()
        cp2 = pltpu.make_async_copy(oval, val_hbm.at[pl.ds(base * KS, RPT * KS)],
                                    sem_out)
        cp2.start()
        pltpu.make_async_copy(oidx, idx_hbm.at[pl.ds(base * KS, RPT * KS)],
                              sem_out).wait()
        cp2.wait()

    return sel_kernel(d2)


def _edge_weights(fk_pad, idxf, validf, wv16):
    mesh = plsc.VectorSubcoreMesh(core_axis_name="c", subcore_axis_name="s")
    cp = pltpu.CompilerParams()
    if "needs_layout_passes" in pltpu.CompilerParams.__dataclass_fields__:
        cp = dataclasses.replace(cp, needs_layout_passes=False)

    @functools.partial(
        pl.kernel,
        compiler_params=cp,
        out_type=jax.ShapeDtypeStruct((NP * KS,), jnp.float32),
        mesh=mesh,
        scratch_types=[
            pltpu.VMEM((RPT * KS,), jnp.int32),
            pltpu.VMEM((RPT * KS,), jnp.float32),
            pltpu.VMEM((RPT * KS,), jnp.float32),
            pltpu.VMEM((2 * KS * FK,), jnp.float32),
            pltpu.VMEM((2 * FK,), jnp.float32),
            pltpu.VMEM((16,), jnp.float32),
            pltpu.SemaphoreType.DMA,
            pltpu.SemaphoreType.DMA,
        ],
    )
    def wk_kernel(fk_hbm, idx_hbm, val_hbm, wv_hbm, out_hbm,
                  idxb, valb, wb, gbuf, ownb, wvb, sem_s, sem_g):
        wid = lax.axis_index("s") * 2 + lax.axis_index("c")
        base = wid * RPT
        iota16 = lax.iota(jnp.int32, 16)
        lane0 = iota16 == 0

        def bc(x):
            return jnp.broadcast_to(x, (16,))

        pltpu.sync_copy(idx_hbm.at[pl.ds(base * KS, RPT * KS)], idxb)
        pltpu.sync_copy(val_hbm.at[pl.ds(base * KS, RPT * KS)], valb)
        pltpu.sync_copy(wv_hbm, wvb)
        wv = wvb[pl.ds(0, 16)]

        def gather_row(r, slot):
            cols0 = idxb[pl.ds(r * KS, 16)]
            cols1 = idxb[pl.ds(r * KS + 16, 16)]
            for j in range(KS):
                col = cols0[j] if j < 16 else cols1[j - 16]
                pltpu.make_async_copy(
                    fk_hbm.at[pl.ds(col * FK, FK)],
                    gbuf.at[pl.ds((slot * KS + j) * FK, FK)], sem_g).start()
            pltpu.make_async_copy(
                fk_hbm.at[pl.ds((base + r) * FK, FK)],
                ownb.at[pl.ds(slot * FK, FK)], sem_s).start()

        def wait_row(slot):
            for j in range(KS):
                pltpu.make_async_copy(
                    fk_hbm.at[pl.ds(0, FK)],
                    gbuf.at[pl.ds(j * FK, FK)], sem_g).wait()
            pltpu.make_async_copy(
                fk_hbm.at[pl.ds(0, FK)],
                ownb.at[pl.ds(0, FK)], sem_s).wait()

        def process(r, slot):
            own = [ownb[pl.ds(slot * FK + c * 16, 16)]
                   for c in range(FK // 16)]

            def edge_body(j, _):
                accs = []
                for k in range(NK):
                    a = None
                    for c4 in range(HC // 16):
                        c = k * (HC // 16) + c4
                        dv = gbuf[pl.ds((slot * KS + j) * FK + c * 16, 16)] - own[c]
                        sq = dv * dv
                        a = sq if a is None else a + sq
                    accs.append(jnp.sum(a))
                dvec = jnp.zeros((16,), jnp.float32)
                for k in range(NK):
                    dvec = jnp.where(iota16 == k, bc(accs[k]), dvec)
                ex = jnp.exp(-dvec)
                wsum = jnp.sum(ex * wv)
                vscal = plsc.load_gather(valb, [bc(r * KS + j)])
                wval = bc(wsum) * vscal
                plsc.store_scatter(wb, [bc(r * KS + j)], wval, mask=lane0)
                return 0

            lax.fori_loop(0, KS, edge_body, 0)

        gather_row(0, 0)
        gather_row(1, 1)

        @pl.loop(0, RPT, step=2)
        def _(g):
            wait_row(0)
            process(g, 0)

            @pl.when(g + 2 < RPT)
            def _():
                gather_row(g + 2, 0)

            wait_row(1)
            process(g + 1, 1)

            @pl.when(g + 3 < RPT)
            def _():
                gather_row(g + 3, 1)

        pltpu.sync_copy(wb, out_hbm.at[pl.ds(base * KS, RPT * KS)])

    return wk_kernel(fk_pad, idxf, validf, wv16)


def _gather_sum(q, idxf, wf):
    mesh = plsc.VectorSubcoreMesh(core_axis_name="c", subcore_axis_name="s")
    cp = pltpu.CompilerParams()
    if "needs_layout_passes" in pltpu.CompilerParams.__dataclass_fields__:
        cp = dataclasses.replace(cp, needs_layout_passes=False)

    @functools.partial(
        pl.kernel,
        compiler_params=cp,
        out_type=jax.ShapeDtypeStruct((NP * NC,), jnp.float32),
        mesh=mesh,
        scratch_types=[
            pltpu.VMEM((RPT * KS,), jnp.int32),
            pltpu.VMEM((RPT * KS,), jnp.float32),
            pltpu.VMEM((RPT * NC,), jnp.float32),
            pltpu.VMEM((2 * KS * NC,), jnp.float32),
            pltpu.SemaphoreType.DMA,
            pltpu.SemaphoreType.DMA,
        ],
    )
    def gs_kernel(q_hbm, idx_hbm, w_hbm, out_hbm,
                  idxb, wb, sb, gbuf, sem_s, sem_g):
        wid = lax.axis_index("s") * 2 + lax.axis_index("c")
        base = wid * RPT

        def gather_row(r, slot):
            cols0 = idxb[pl.ds(r * KS, 16)]
            cols1 = idxb[pl.ds(r * KS + 16, 16)]
            for j in range(KS):
                col = cols0[j] if j < 16 else cols1[j - 16]
                pltpu.make_async_copy(
                    q_hbm.at[pl.ds(col * NC, NC)],
                    gbuf.at[pl.ds((slot * KS + j) * NC, NC)], sem_g).start()

        def wait_row(slot):
            for j in range(KS):
                pltpu.make_async_copy(
                    q_hbm.at[pl.ds(0, NC)],
                    gbuf.at[pl.ds(j * NC, NC)], sem_g).wait()

        def process(r, slot):
            acc0 = jnp.zeros((16,), jnp.float32)
            acc1 = jnp.zeros((16,), jnp.float32)
            for j in range(KS):
                wj = plsc.load_gather(
                    wb, [jnp.broadcast_to(r * KS + j, (16,))])
                gb = (slot * KS + j) * NC
                acc0 = acc0 + wj * gbuf[pl.ds(gb, 16)]
                acc1 = acc1 + wj * gbuf[pl.ds(gb + 16, 16)]
            sb[pl.ds(r * NC, 16)] = acc0
            sb[pl.ds(r * NC + 16, 16)] = acc1

        pltpu.sync_copy(idx_hbm.at[pl.ds(base * KS, RPT * KS)], idxb)
        pltpu.sync_copy(w_hbm.at[pl.ds(base * KS, RPT * KS)], wb)
        gather_row(0, 0)
        gather_row(1, 1)

        @pl.loop(0, RPT, step=2)
        def _(g):
            wait_row(0)
            process(g, 0)

            @pl.when(g + 2 < RPT)
            def _():
                gather_row(g + 2, 0)

            wait_row(1)
            process(g + 1, 1)

            @pl.when(g + 3 < RPT)
            def _():
                gather_row(g + 3, 1)

        pltpu.sync_copy(sb, out_hbm.at[pl.ds(base * NC, RPT * NC)])

    return gs_kernel(q, idxf, wf)


def _fk_body(f_ref, fr_ref, out_ref):
    out_ref[...] = lax.dot_general(
        f_ref[...], fr_ref[...], (((1,), (0,)), ((), ())),
        preferred_element_type=jnp.float32,
        precision=lax.Precision.HIGHEST)


def _fk_matmul(f, Fr):
    rb = 400
    return pl.pallas_call(
        _fk_body,
        grid=(N // rb,),
        in_specs=[
            pl.BlockSpec((rb, EC), lambda i: (i, 0)),
            pl.BlockSpec((EC, FK), lambda i: (0, 0)),
        ],
        out_specs=pl.BlockSpec((rb, FK), lambda i: (i, 0)),
        out_shape=jax.ShapeDtypeStruct((N, FK), jnp.float32),
    )(f, Fr)


def _step_body(s_ref, u_ref, c_ref, out_ref):
    q = lax.dot_general(
        s_ref[...], c_ref[...], (((1,), (0,)), ((), ())),
        preferred_element_type=jnp.float32)
    z = -u_ref[...] - q
    z = z - jnp.max(z, axis=-1, keepdims=True)
    e = jnp.exp(z)
    out_ref[...] = e / jnp.sum(e, axis=-1, keepdims=True)


def _crf_step(sred, u, C):
    rb = 400
    return pl.pallas_call(
        _step_body,
        grid=(N // rb,),
        in_specs=[
            pl.BlockSpec((rb, NC), lambda i: (i, 0)),
            pl.BlockSpec((rb, NC), lambda i: (i, 0)),
            pl.BlockSpec((NC, NC), lambda i: (0, 0)),
        ],
        out_specs=pl.BlockSpec((rb, NC), lambda i: (i, 0)),
        out_shape=jax.ShapeDtypeStruct((N, NC), jnp.float32),
    )(sred, u, C)


def kernel(pos, p, f, F, W, C):
    pos8 = jnp.zeros((NP, 8), jnp.float32).at[:N, :3].set(pos)
    pos8 = pos8.at[N:, 0].set(100.0)
    pos8t = jnp.zeros((8, CP), jnp.float32).at[:3, :N].set(pos.T)
    pos8t = pos8t.at[0, N:].set(200.0)

    d2 = _d2_matrix(pos8, pos8t)
    idxf, validf = _knn_select(d2)

    Fr = jnp.transpose(F, (1, 0, 2)).reshape(EC, FK)
    fk = _fk_matmul(f, Fr)
    fk_pad = jnp.zeros((NP, FK), jnp.float32).at[:N].set(fk).reshape(-1)
    wv16 = jnp.zeros((16,), jnp.float32).at[:NK].set(W[:, 0])

    wf = _edge_weights(fk_pad, idxf, validf, wv16)

    u = -jnp.log(p)
    q = p
    for _ in range(STEPS):
        sred = _gather_sum(q.reshape(-1), idxf, wf).reshape(NP, NC)[:N]
        q = _crf_step(sred, u, C)
    return q


# speedup vs baseline: 1.0305x; 1.0305x over previous
import dataclasses
import functools

import jax
import jax.numpy as jnp
from jax import lax
from jax.experimental import pallas as pl
from jax.experimental.pallas import tpu as pltpu
from jax.experimental.pallas import tpu_sc as plsc

N = 10000
NC = 32
EC = 128
HC = 64
NK = 5
R2 = 0.2 * 0.2
KS = 32
STEPS = 5

FK = NK * HC
NP = 10240
CP = 10240
NTILES = 32
RPT = NP // NTILES
CAP = 512
CM = CAP // 16
NCHUNK = CP // 16
BIG = 1e9


def _d2_body(a_ref, b_ref, out_ref):
    i = pl.program_id(0)
    j = pl.program_id(1)
    rb = out_ref.shape[0]
    cb = out_ref.shape[1]
    acc = None
    for co in range(3):
        a = a_ref[:, co:co + 1]
        b = b_ref[co:co + 1, :]
        d = a - b
        acc = d * d if acc is None else acc + d * d
    rid = lax.broadcasted_iota(jnp.int32, (rb, cb), 0) + i * rb
    cid = lax.broadcasted_iota(jnp.int32, (rb, cb), 1) + j * cb
    out_ref[...] = jnp.where(rid == cid, BIG, acc)


def _d2_matrix(pos8, pos8t):
    rb, cb = 640, 2560
    return pl.pallas_call(
        _d2_body,
        grid=(NP // rb, CP // cb),
        in_specs=[
            pl.BlockSpec((rb, 8), lambda i, j: (i, 0)),
            pl.BlockSpec((8, cb), lambda i, j: (0, j)),
        ],
        out_specs=pl.BlockSpec((rb, cb), lambda i, j: (i, j)),
        out_shape=jax.ShapeDtypeStruct((NP, CP), jnp.float32),
    )(pos8, pos8t)


def _knn_select(d2):
    mesh = plsc.VectorSubcoreMesh(core_axis_name="c", subcore_axis_name="s")

    cp = pltpu.CompilerParams()
    if "needs_layout_passes" in pltpu.CompilerParams.__dataclass_fields__:
        cp = dataclasses.replace(cp, needs_layout_passes=False)

    @functools.partial(
        pl.kernel,
        compiler_params=cp,
        out_type=[
            jax.ShapeDtypeStruct((NP * KS,), jnp.int32),
            jax.ShapeDtypeStruct((NP * KS,), jnp.float32),
        ],
        mesh=mesh,
        scratch_types=[
            pltpu.VMEM((2, CP), jnp.float32),
            pltpu.VMEM((CAP,), jnp.float32),
            pltpu.VMEM((CAP,), jnp.int32),
            pltpu.VMEM((RPT * KS,), jnp.int32),
            pltpu.VMEM((RPT * KS,), jnp.float32),
            pltpu.SemaphoreType.DMA,
            pltpu.SemaphoreType.DMA,
        ],
    )
    def sel_kernel(d2_hbm, idx_hbm, val_hbm, rowbuf, cvals, cidx,
                   oidx, oval, sem_in, sem_out):
        wid = lax.axis_index("s") * 2 + lax.axis_index("c")
        base = wid * RPT
        iota16 = lax.iota(jnp.int32, 16)
        inf16 = jnp.full((16,), BIG, jnp.float32)
        zero16 = jnp.zeros((16,), jnp.int32)

        def bc(x):
            return jnp.broadcast_to(x, (16,))

        def process(buf, lrow):
            @pl.loop(0, CM)
            def _(jc):
                cvals[pl.ds(jc * 16, 16)] = inf16
                cidx[pl.ds(jc * 16, 16)] = iota16 + jc * 16

            UNR = 8

            def chunk_body(g, cntv):
                vs, ms, pss, pcs = [], [], [], []
                for uu in range(UNR):
                    c = g * UNR + uu
                    v = rowbuf[buf, pl.ds(c * 16, 16)]
                    m = v < R2
                    vs.append(v)
                    ms.append(m)
                    pss.append(jnp.cumsum(m.astype(jnp.int32)))
                    pcs.append(plsc.all_reduce_population_count(m))
                for uu in range(UNR):
                    c = g * UNR + uu
                    pos = cntv + pss[uu] - 1
                    mm = ms[uu] & (pos < CAP)
                    colv = iota16 + c * 16
                    plsc.store_scatter(cvals, [pos], vs[uu], mask=mm)
                    plsc.store_scatter(cidx, [pos], colv, mask=mm)
                    cntv = cntv + pcs[uu]
                return cntv

            lax.fori_loop(0, NCHUNK // UNR, chunk_body,
                          jnp.zeros((16,), jnp.int32))

            def cmin16(cbase):
                acc = inf16
                for l in range(16):
                    g = plsc.load_gather(cvals, [(iota16 + cbase) * 16 + l])
                    acc = jnp.minimum(acc, g)
                return acc

            cm0 = cmin16(0)
            cm1 = cmin16(16)

            def ext_body(t, carry):
                cm0, cm1, sel, selv = carry
                minv = jnp.min(jnp.minimum(cm0, cm1))
                p0 = jnp.where(cm0 == minv, iota16, 64)
                p1 = jnp.where(cm1 == minv, iota16 + 16, 64)
                cstar = jnp.min(jnp.minimum(p0, p1))
                off = cstar * 16
                v = cvals[pl.ds(off, 16)]
                lane = jnp.min(jnp.where(v == minv, iota16, 15))
                col = plsc.load_gather(cidx, [bc(off + lane)])
                validv = jnp.where(bc(minv) < R2, 1.0, 0.0)
                tl = t % 16
                sel = jnp.where(iota16 == tl, col, sel)
                selv = jnp.where(iota16 == tl, validv, selv)
                vupd = jnp.where(iota16 == lane, BIG, v)
                cvals[pl.ds(off, 16)] = vupd
                nm = bc(jnp.min(vupd))
                cm0 = jnp.where(iota16 == cstar, nm, cm0)
                cm1 = jnp.where(iota16 + 16 == cstar, nm, cm1)
                return cm0, cm1, sel, selv

            obase = lrow * KS
            cm0, cm1, sel, selv = lax.fori_loop(
                0, 16, ext_body, (cm0, cm1, zero16, jnp.zeros((16,), jnp.float32)))
            oidx[pl.ds(obase, 16)] = sel
            oval[pl.ds(obase, 16)] = selv
            cm0, cm1, sel, selv = lax.fori_loop(
                16, 32, ext_body, (cm0, cm1, zero16, jnp.zeros((16,), jnp.float32)))
            oidx[pl.ds(obase + 16, 16)] = sel
            oval[pl.ds(obase + 16, 16)] = selv

        pltpu.make_async_copy(d2_hbm.at[base], rowbuf.at[0], sem_in).start()
        pltpu.make_async_copy(d2_hbm.at[base + 1], rowbuf.at[1], sem_in).start()

        @pl.loop(0, RPT, step=2)
        def _(g):
            pltpu.make_async_copy(d2_hbm.at[base + g],
                                  rowbuf.at[0], sem_in).wait()
            process(0, g)

            @pl.when(g + 2 < RPT)
            def _():
                pltpu.make_async_copy(d2_hbm.at[base + g + 2],
                                      rowbuf.at[0], sem_in).start()

            pltpu.make_async_copy(d2_hbm.at[base + g + 1],
                                  rowbuf.at[1], sem_in).wait()
            process(1, g + 1)

            @pl.when(g + 3 < RPT)
            def _():
                pltpu.make_async_copy(d2_hbm.at[base + g + 3],
                                      rowbuf.at[1], sem_in).start()

        pltpu.make_async_copy(oidx, idx_hbm.at[pl.ds(base * KS, RPT * KS)],
                              sem_out).start()
        cp2 = pltpu.make_async_copy(oval, val_hbm.at[pl.ds(base * KS, RPT * KS)],
                                    sem_out)
        cp2.start()
        pltpu.make_async_copy(oidx, idx_hbm.at[pl.ds(base * KS, RPT * KS)],
                              sem_out).wait()
        cp2.wait()

    return sel_kernel(d2)


def _edge_weights(fk_pad, idxf, validf, wv16):
    mesh = plsc.VectorSubcoreMesh(core_axis_name="c", subcore_axis_name="s")
    cp = pltpu.CompilerParams()
    if "needs_layout_passes" in pltpu.CompilerParams.__dataclass_fields__:
        cp = dataclasses.replace(cp, needs_layout_passes=False)

    @functools.partial(
        pl.kernel,
        compiler_params=cp,
        out_type=jax.ShapeDtypeStruct((NP * KS,), jnp.float32),
        mesh=mesh,
        scratch_types=[
            pltpu.VMEM((RPT * KS,), jnp.int32),
            pltpu.VMEM((RPT * KS,), jnp.float32),
            pltpu.VMEM((RPT * KS,), jnp.float32),
            pltpu.VMEM((2 * KS * FK,), jnp.float32),
            pltpu.VMEM((2 * FK,), jnp.float32),
            pltpu.VMEM((16,), jnp.float32),
            pltpu.SemaphoreType.DMA,
            pltpu.SemaphoreType.DMA,
        ],
    )
    def wk_kernel(fk_hbm, idx_hbm, val_hbm, wv_hbm, out_hbm,
                  idxb, valb, wb, gbuf, ownb, wvb, sem_s, sem_g):
        wid = lax.axis_index("s") * 2 + lax.axis_index("c")
        base = wid * RPT
        iota16 = lax.iota(jnp.int32, 16)
        lane0 = iota16 == 0

        def bc(x):
            return jnp.broadcast_to(x, (16,))

        pltpu.sync_copy(idx_hbm.at[pl.ds(base * KS, RPT * KS)], idxb)
        pltpu.sync_copy(val_hbm.at[pl.ds(base * KS, RPT * KS)], valb)
        pltpu.sync_copy(wv_hbm, wvb)
        wv = wvb[pl.ds(0, 16)]

        def gather_row(r, slot):
            cols0 = idxb[pl.ds(r * KS, 16)]
            cols1 = idxb[pl.ds(r * KS + 16, 16)]
            for j in range(KS):
                col = cols0[j] if j < 16 else cols1[j - 16]
                pltpu.make_async_copy(
                    fk_hbm.at[pl.ds(col * FK, FK)],
                    gbuf.at[pl.ds((slot * KS + j) * FK, FK)], sem_g).start()
            pltpu.make_async_copy(
                fk_hbm.at[pl.ds((base + r) * FK, FK)],
                ownb.at[pl.ds(slot * FK, FK)], sem_s).start()

        def wait_row(slot):
            for j in range(KS):
                pltpu.make_async_copy(
                    fk_hbm.at[pl.ds(0, FK)],
                    gbuf.at[pl.ds(j * FK, FK)], sem_g).wait()
            pltpu.make_async_copy(
                fk_hbm.at[pl.ds(0, FK)],
                ownb.at[pl.ds(0, FK)], sem_s).wait()

        def process(r, slot):
            own = [ownb[pl.ds(slot * FK + c * 16, 16)]
                   for c in range(FK // 16)]

            def edge_body(j, _):
                accs = []
                for k in range(NK):
                    a = None
                    for c4 in range(HC // 16):
                        c = k * (HC // 16) + c4
                        dv = gbuf[pl.ds((slot * KS + j) * FK + c * 16, 16)] - own[c]
                        sq = dv * dv
                        a = sq if a is None else a + sq
                    accs.append(jnp.sum(a))
                dvec = jnp.zeros((16,), jnp.float32)
                for k in range(NK):
                    dvec = jnp.where(iota16 == k, bc(accs[k]), dvec)
                ex = jnp.exp(-dvec)
                wsum = jnp.sum(ex * wv)
                vscal = plsc.load_gather(valb, [bc(r * KS + j)])
                wval = bc(wsum) * vscal
                plsc.store_scatter(wb, [bc(r * KS + j)], wval, mask=lane0)
                return 0

            lax.fori_loop(0, KS, edge_body, 0)

        gather_row(0, 0)
        gather_row(1, 1)

        @pl.loop(0, RPT, step=2)
        def _(g):
            wait_row(0)
            process(g, 0)

            @pl.when(g + 2 < RPT)
            def _():
                gather_row(g + 2, 0)

            wait_row(1)
            process(g + 1, 1)

            @pl.when(g + 3 < RPT)
            def _():
                gather_row(g + 3, 1)

        pltpu.sync_copy(wb, out_hbm.at[pl.ds(base * KS, RPT * KS)])

    return wk_kernel(fk_pad, idxf, validf, wv16)


def _gather_sum(q, idxf, wf):
    mesh = plsc.VectorSubcoreMesh(core_axis_name="c", subcore_axis_name="s")
    cp = pltpu.CompilerParams()
    if "needs_layout_passes" in pltpu.CompilerParams.__dataclass_fields__:
        cp = dataclasses.replace(cp, needs_layout_passes=False)

    @functools.partial(
        pl.kernel,
        compiler_params=cp,
        out_type=jax.ShapeDtypeStruct((NP * NC,), jnp.float32),
        mesh=mesh,
        scratch_types=[
            pltpu.VMEM((RPT * KS,), jnp.int32),
            pltpu.VMEM((RPT * KS,), jnp.float32),
            pltpu.VMEM((RPT * NC,), jnp.float32),
            pltpu.VMEM((4 * KS * NC,), jnp.float32),
            pltpu.SemaphoreType.DMA,
            pltpu.SemaphoreType.DMA,
        ],
    )
    def gs_kernel(q_hbm, idx_hbm, w_hbm, out_hbm,
                  idxb, wb, sb, gbuf, sem_s, sem_g):
        wid = lax.axis_index("s") * 2 + lax.axis_index("c")
        base = wid * RPT

        def gather_row(r, slot):
            cols0 = idxb[pl.ds(r * KS, 16)]
            cols1 = idxb[pl.ds(r * KS + 16, 16)]
            for j in range(KS):
                col = cols0[j] if j < 16 else cols1[j - 16]
                pltpu.make_async_copy(
                    q_hbm.at[pl.ds(col * NC, NC)],
                    gbuf.at[pl.ds((slot * KS + j) * NC, NC)], sem_g).start()

        def wait_row(slot):
            for j in range(KS):
                pltpu.make_async_copy(
                    q_hbm.at[pl.ds(0, NC)],
                    gbuf.at[pl.ds(j * NC, NC)], sem_g).wait()

        def process(r, slot):
            acc0 = jnp.zeros((16,), jnp.float32)
            acc1 = jnp.zeros((16,), jnp.float32)
            for j in range(KS):
                wj = plsc.load_gather(
                    wb, [jnp.broadcast_to(r * KS + j, (16,))])
                gb = (slot * KS + j) * NC
                acc0 = acc0 + wj * gbuf[pl.ds(gb, 16)]
                acc1 = acc1 + wj * gbuf[pl.ds(gb + 16, 16)]
            sb[pl.ds(r * NC, 16)] = acc0
            sb[pl.ds(r * NC + 16, 16)] = acc1

        pltpu.sync_copy(idx_hbm.at[pl.ds(base * KS, RPT * KS)], idxb)
        pltpu.sync_copy(w_hbm.at[pl.ds(base * KS, RPT * KS)], wb)
        for sl in range(4):
            gather_row(sl, sl)

        @pl.loop(0, RPT, step=4)
        def _(g):
            for sl in range(4):
                wait_row(sl)
                process(g + sl, sl)

                @pl.when(g + sl + 4 < RPT)
                def _():
                    gather_row(g + sl + 4, sl)

        pltpu.sync_copy(sb, out_hbm.at[pl.ds(base * NC, RPT * NC)])

    return gs_kernel(q, idxf, wf)


def _fk_body(f_ref, fr_ref, out_ref):
    out_ref[...] = lax.dot_general(
        f_ref[...], fr_ref[...], (((1,), (0,)), ((), ())),
        preferred_element_type=jnp.float32,
        precision=lax.Precision.HIGHEST)


def _fk_matmul(f, Fr):
    rb = 400
    return pl.pallas_call(
        _fk_body,
        grid=(N // rb,),
        in_specs=[
            pl.BlockSpec((rb, EC), lambda i: (i, 0)),
            pl.BlockSpec((EC, FK), lambda i: (0, 0)),
        ],
        out_specs=pl.BlockSpec((rb, FK), lambda i: (i, 0)),
        out_shape=jax.ShapeDtypeStruct((N, FK), jnp.float32),
    )(f, Fr)


def _step_body(s_ref, u_ref, c_ref, out_ref):
    q = lax.dot_general(
        s_ref[...], c_ref[...], (((1,), (0,)), ((), ())),
        preferred_element_type=jnp.float32)
    z = -u_ref[...] - q
    z = z - jnp.max(z, axis=-1, keepdims=True)
    e = jnp.exp(z)
    out_ref[...] = e / jnp.sum(e, axis=-1, keepdims=True)


def _crf_step(sred, u, C):
    rb = 400
    return pl.pallas_call(
        _step_body,
        grid=(N // rb,),
        in_specs=[
            pl.BlockSpec((rb, NC), lambda i: (i, 0)),
            pl.BlockSpec((rb, NC), lambda i: (i, 0)),
            pl.BlockSpec((NC, NC), lambda i: (0, 0)),
        ],
        out_specs=pl.BlockSpec((rb, NC), lambda i: (i, 0)),
        out_shape=jax.ShapeDtypeStruct((N, NC), jnp.float32),
    )(sred, u, C)


def kernel(pos, p, f, F, W, C):
    pos8 = jnp.zeros((NP, 8), jnp.float32).at[:N, :3].set(pos)
    pos8 = pos8.at[N:, 0].set(100.0)
    pos8t = jnp.zeros((8, CP), jnp.float32).at[:3, :N].set(pos.T)
    pos8t = pos8t.at[0, N:].set(200.0)

    d2 = _d2_matrix(pos8, pos8t)
    idxf, validf = _knn_select(d2)

    Fr = jnp.transpose(F, (1, 0, 2)).reshape(EC, FK)
    fk = _fk_matmul(f, Fr)
    fk_pad = jnp.zeros((NP, FK), jnp.float32).at[:N].set(fk).reshape(-1)
    wv16 = jnp.zeros((16,), jnp.float32).at[:NK].set(W[:, 0])

    wf = _edge_weights(fk_pad, idxf, validf, wv16)

    u = -jnp.log(p)
    q = p
    for _ in range(STEPS):
        sred = _gather_sum(q.reshape(-1), idxf, wf).reshape(NP, NC)[:N]
        q = _crf_step(sred, u, C)
    return q
